# baseline (speedup 1.0000x reference)
import functools
import math

import jax
import jax.numpy as jnp
from jax import lax
from jax.experimental import pallas as pl
from jax.experimental.pallas import tpu as pltpu

K = 8
R = 3
NEG = -1e38
LANES = 128


def _round_body(r, is_last, BT, T, D, NI,
                params_ref, h_ref, *rest):
    if is_last:
        x_ref, gain_ref, bias_ref, out_ref, s_ref, w_ref, m_ref = rest
    else:
        gain_ref, bias_ref, out_ref, s_ref, w_ref, m_ref = rest
    i = pl.program_id(1)
    mix = params_ref[r]
    momentum = params_ref[R]
    scale = params_ref[R + 1]

    q = h_ref[0, pl.ds(i * BT, BT), :]
    rowloc = i * BT + lax.broadcasted_iota(jnp.int32, (BT, 1), 0)
    colsl = lax.broadcasted_iota(jnp.int32, (BT, BT), 1)
    lane = lax.broadcasted_iota(jnp.int32, (BT, LANES), 1)

    def fill(j, carry):
        kblk = h_ref[0, pl.ds(j * BT, BT), :]
        s_ref[:, pl.ds(j * BT, BT)] = lax.dot_general(
            q, kblk, (((1,), (1,)), ((), ())),
            preferred_element_type=jnp.float32)
        return carry

    lax.fori_loop(0, i + 1, fill, 0)

    dsl = pl.ds(i * BT, BT)
    s_ref[:, dsl] = jnp.where(i * BT + colsl <= rowloc, s_ref[:, dsl], NEG)

    def seg_body(j, cv):
        w_ref[...] = s_ref[:, pl.ds(j * BT, BT)]
        for kk in range(K):
            w = w_ref[...]
            m = jnp.max(w, axis=1, keepdims=True)
            cv = jnp.where(lane == j * K + kk, m, cv)
            w_ref[...] = jnp.where(w == m, NEG, w)
        return cv

    cv = lax.fori_loop(0, i + 1, seg_body,
                       jnp.full((BT, LANES), NEG, jnp.float32))

    t8 = None
    for _ in range(K):
        t8 = jnp.max(cv, axis=1, keepdims=True)
        cv = jnp.where(cv == t8, NEG, cv)

    m_ref[...] = jnp.zeros((BT, D), jnp.float32)

    def agg(j, carry):
        sblk = s_ref[:, pl.ds(j * BT, BT)]
        ablk = jnp.where((sblk >= t8) & (j * BT + colsl <= rowloc), 1.0, 0.0)
        hblk = h_ref[0, pl.ds(j * BT, BT), :]
        m_ref[...] += lax.dot_general(
            ablk, hblk, (((1,), (0,)), ((), ())),
            preferred_element_type=jnp.float32)
        return carry

    lax.fori_loop(0, i + 1, agg, 0)

    deg = jnp.minimum(rowloc.astype(jnp.float32) + 1.0, float(K))
    msg = m_ref[...] / deg

    blended = mix * q + (1.0 - mix) * msg
    gb = blended * gain_ref[0] + bias_ref[0]
    act = gb * 0.5 * (1.0 + lax.erf(gb * (1.0 / math.sqrt(2.0))))
    hn = momentum * q + (1.0 - momentum) * act
    if is_last:
        out_ref[0] = (hn - x_ref[0, pl.ds(i * BT, BT), :]) * scale
    else:
        out_ref[0] = hn


def _round_call(r, is_last, h, x, gain_r, bias_r, params, BT=256):
    B, T, D = h.shape
    NI = T // BT
    body = functools.partial(_round_body, r, is_last, BT, T, D, NI)
    in_specs = [
        pl.BlockSpec(memory_space=pltpu.SMEM),
        pl.BlockSpec((1, T, D), lambda b, i: (b, 0, 0)),
    ]
    args = [params, h]
    if is_last:
        in_specs.append(pl.BlockSpec((1, T, D), lambda b, i: (b, 0, 0)))
        args.append(x)
    in_specs += [
        pl.BlockSpec((1, D), lambda b, i: (0, 0)),
        pl.BlockSpec((1, D), lambda b, i: (0, 0)),
    ]
    args += [gain_r, bias_r]
    return pl.pallas_call(
        body,
        grid=(B, NI),
        in_specs=in_specs,
        out_specs=pl.BlockSpec((1, BT, D), lambda b, i: (b, i, 0)),
        out_shape=jax.ShapeDtypeStruct((B, T, D), jnp.float32),
        scratch_shapes=[
            pltpu.VMEM((BT, T), jnp.float32),
            pltpu.VMEM((BT, BT), jnp.float32),
            pltpu.VMEM((BT, D), jnp.float32),
        ],
    )(*args)


def kernel(x, gain, bias, log_mix, log_momentum, log_scale):
    B, T, D = x.shape
    momentum = jax.nn.sigmoid(log_momentum)
    scale = jax.nn.softplus(log_scale) + 0.01
    mix = jax.nn.sigmoid(log_mix)
    params = jnp.concatenate(
        [mix.astype(jnp.float32),
         jnp.stack([momentum, scale]).astype(jnp.float32)])
    h = x
    for r in range(R):
        h = _round_call(r, r == R - 1, h, x,
                        gain[r].reshape(1, D), bias[r].reshape(1, D), params)
    return h


# speedup vs baseline: 1.6540x; 1.6540x over previous
import functools
import math

import jax
import jax.numpy as jnp
from jax.experimental import pallas as pl
from jax.experimental.pallas import tpu as pltpu

K = 8
R = 3
NEG = -1e38


def _round_body(r, is_last, BT, T, D, NI,
                params_ref, h_ref, *rest):
    if is_last:
        x_ref, gain_ref, bias_ref, out_ref, s_ref, m_ref = rest
    else:
        gain_ref, bias_ref, out_ref, s_ref, m_ref = rest
    i = pl.program_id(1)
    mix = params_ref[r]
    momentum = params_ref[R]
    scale = params_ref[R + 1]

    q = h_ref[0, pl.ds(i * BT, BT), :]

    def fill(j, carry):
        kblk = h_ref[0, pl.ds(j * BT, BT), :]
        s_ref[:, pl.ds(j * BT, BT)] = jax.lax.dot_general(
            q, kblk, (((1,), (1,)), ((), ())),
            preferred_element_type=jnp.float32)
        return carry

    jax.lax.fori_loop(0, i + 1, fill, 0)

    rows = i * BT + jax.lax.broadcasted_iota(jnp.int32, (BT, T), 0)
    cols = jax.lax.broadcasted_iota(jnp.int32, (BT, T), 1)
    causal = cols <= rows
    s_ref[...] = jnp.where(causal, s_ref[...], NEG)

    for _ in range(K):
        s = s_ref[...]
        m = jnp.max(s, axis=1, keepdims=True)
        s_ref[...] = jnp.where(s == m, NEG, s)

    s_ref[...] = jnp.where(causal & (s_ref[...] == NEG), 1.0, 0.0)

    m_ref[...] = jnp.zeros((BT, D), jnp.float32)

    def agg(j, carry):
        ablk = s_ref[:, pl.ds(j * BT, BT)]
        hblk = h_ref[0, pl.ds(j * BT, BT), :]
        m_ref[...] += jax.lax.dot_general(
            ablk, hblk, (((1,), (0,)), ((), ())),
            preferred_element_type=jnp.float32)
        return carry

    jax.lax.fori_loop(0, i + 1, agg, 0)

    row1 = i * BT + jax.lax.broadcasted_iota(jnp.int32, (BT, 1), 0)
    deg = jnp.minimum(row1.astype(jnp.float32) + 1.0, float(K))
    msg = m_ref[...] / deg

    blended = mix * q + (1.0 - mix) * msg
    gb = blended * gain_ref[0] + bias_ref[0]
    act = gb * 0.5 * (1.0 + jax.lax.erf(gb * (1.0 / math.sqrt(2.0))))
    hn = momentum * q + (1.0 - momentum) * act
    if is_last:
        out_ref[0] = (hn - x_ref[0, pl.ds(i * BT, BT), :]) * scale
    else:
        out_ref[0] = hn


def _round_call(r, is_last, h, x, gain_r, bias_r, params, BT=512):
    B, T, D = h.shape
    NI = T // BT
    body = functools.partial(_round_body, r, is_last, BT, T, D, NI)
    in_specs = [
        pl.BlockSpec(memory_space=pltpu.SMEM),
        pl.BlockSpec((1, T, D), lambda b, i: (b, 0, 0)),
    ]
    args = [params, h]
    if is_last:
        in_specs.append(pl.BlockSpec((1, T, D), lambda b, i: (b, 0, 0)))
        args.append(x)
    in_specs += [
        pl.BlockSpec((1, D), lambda b, i: (0, 0)),
        pl.BlockSpec((1, D), lambda b, i: (0, 0)),
    ]
    args += [gain_r, bias_r]
    return pl.pallas_call(
        body,
        grid=(B, NI),
        in_specs=in_specs,
        out_specs=pl.BlockSpec((1, BT, D), lambda b, i: (b, i, 0)),
        out_shape=jax.ShapeDtypeStruct((B, T, D), jnp.float32),
        scratch_shapes=[
            pltpu.VMEM((BT, T), jnp.float32),
            pltpu.VMEM((BT, D), jnp.float32),
        ],
    )(*args)


def kernel(x, gain, bias, log_mix, log_momentum, log_scale):
    B, T, D = x.shape
    momentum = jax.nn.sigmoid(log_momentum)
    scale = jax.nn.softplus(log_scale) + 0.01
    mix = jax.nn.sigmoid(log_mix)
    params = jnp.concatenate(
        [mix.astype(jnp.float32),
         jnp.stack([momentum, scale]).astype(jnp.float32)])
    h = x
    for r in range(R):
        h = _round_call(r, r == R - 1, h, x,
                        gain[r].reshape(1, D), bias[r].reshape(1, D), params)
    return h


# speedup vs baseline: 1.7034x; 1.0299x over previous
import functools
import math

import jax
import jax.numpy as jnp
from jax.experimental import pallas as pl
from jax.experimental.pallas import tpu as pltpu

K = 8
R = 3
NEG = -1e38


def _round_body(r, is_last, BT, T, D, NI,
                params_ref, h_ref, *rest):
    if is_last:
        x_ref, gain_ref, bias_ref, out_ref, s_ref, m_ref = rest
    else:
        gain_ref, bias_ref, out_ref, s_ref, m_ref = rest
    i = pl.program_id(1)
    mix = params_ref[r]
    momentum = params_ref[R]
    scale = params_ref[R + 1]

    q = h_ref[0, pl.ds(i * BT, BT), :]

    def fill(j, carry):
        kblk = h_ref[0, pl.ds(j * BT, BT), :]
        s_ref[:, pl.ds(j * BT, BT)] = jax.lax.dot_general(
            q, kblk, (((1,), (1,)), ((), ())),
            preferred_element_type=jnp.float32)
        return carry

    jax.lax.fori_loop(0, i + 1, fill, 0)

    rows = i * BT + jax.lax.broadcasted_iota(jnp.int32, (BT, T), 0)
    cols = jax.lax.broadcasted_iota(jnp.int32, (BT, T), 1)
    causal = cols <= rows
    s_ref[...] = jnp.where(causal, s_ref[...], NEG)

    for _ in range(K):
        s = s_ref[...]
        m = jnp.max(s, axis=1, keepdims=True)
        s_ref[...] = jnp.where(s == m, NEG, s)

    s_ref[...] = jnp.where(causal & (s_ref[...] == NEG), 1.0, 0.0)

    m_ref[...] = jnp.zeros((BT, D), jnp.float32)

    def agg(j, carry):
        ablk = s_ref[:, pl.ds(j * BT, BT)]
        hblk = h_ref[0, pl.ds(j * BT, BT), :]
        m_ref[...] += jax.lax.dot_general(
            ablk, hblk, (((1,), (0,)), ((), ())),
            preferred_element_type=jnp.float32)
        return carry

    jax.lax.fori_loop(0, i + 1, agg, 0)

    row1 = i * BT + jax.lax.broadcasted_iota(jnp.int32, (BT, 1), 0)
    deg = jnp.minimum(row1.astype(jnp.float32) + 1.0, float(K))
    msg = m_ref[...] / deg

    blended = mix * q + (1.0 - mix) * msg
    gb = blended * gain_ref[0] + bias_ref[0]
    act = gb * 0.5 * (1.0 + jax.lax.erf(gb * (1.0 / math.sqrt(2.0))))
    hn = momentum * q + (1.0 - momentum) * act
    if is_last:
        out_ref[0] = (hn - x_ref[0, pl.ds(i * BT, BT), :]) * scale
    else:
        out_ref[0] = hn


def _round_call(r, is_last, h, x, gain_r, bias_r, params, BT=1024):
    B, T, D = h.shape
    NI = T // BT
    body = functools.partial(_round_body, r, is_last, BT, T, D, NI)
    in_specs = [
        pl.BlockSpec(memory_space=pltpu.SMEM),
        pl.BlockSpec((1, T, D), lambda b, i: (b, 0, 0)),
    ]
    args = [params, h]
    if is_last:
        in_specs.append(pl.BlockSpec((1, T, D), lambda b, i: (b, 0, 0)))
        args.append(x)
    in_specs += [
        pl.BlockSpec((1, D), lambda b, i: (0, 0)),
        pl.BlockSpec((1, D), lambda b, i: (0, 0)),
    ]
    args += [gain_r, bias_r]
    return pl.pallas_call(
        body,
        grid=(B, NI),
        in_specs=in_specs,
        out_specs=pl.BlockSpec((1, BT, D), lambda b, i: (b, i, 0)),
        out_shape=jax.ShapeDtypeStruct((B, T, D), jnp.float32),
        scratch_shapes=[
            pltpu.VMEM((BT, T), jnp.float32),
            pltpu.VMEM((BT, D), jnp.float32),
        ],
    )(*args)


def kernel(x, gain, bias, log_mix, log_momentum, log_scale):
    B, T, D = x.shape
    momentum = jax.nn.sigmoid(log_momentum)
    scale = jax.nn.softplus(log_scale) + 0.01
    mix = jax.nn.sigmoid(log_mix)
    params = jnp.concatenate(
        [mix.astype(jnp.float32),
         jnp.stack([momentum, scale]).astype(jnp.float32)])
    h = x
    for r in range(R):
        h = _round_call(r, r == R - 1, h, x,
                        gain[r].reshape(1, D), bias[r].reshape(1, D), params)
    return h


# speedup vs baseline: 1.7045x; 1.0006x over previous
import functools
import math

import jax
import jax.numpy as jnp
from jax.experimental import pallas as pl
from jax.experimental.pallas import tpu as pltpu

K = 8
R = 3
NEG = -1e38


def _round_body(r, is_last, BT, T, D, NI,
                params_ref, h_ref, *rest):
    if is_last:
        x_ref, gain_ref, bias_ref, out_ref, s_ref, m_ref = rest
    else:
        gain_ref, bias_ref, out_ref, s_ref, m_ref = rest
    i = pl.program_id(1)
    mix = params_ref[r]
    momentum = params_ref[R]
    scale = params_ref[R + 1]

    q = h_ref[0, pl.ds(i * BT, BT), :]

    def fill(j, carry):
        kblk = h_ref[0, pl.ds(j * BT, BT), :]
        s_ref[:, pl.ds(j * BT, BT)] = jax.lax.dot_general(
            q, kblk, (((1,), (1,)), ((), ())),
            preferred_element_type=jnp.float32)
        return carry

    jax.lax.fori_loop(0, i + 1, fill, 0)

    rows = i * BT + jax.lax.broadcasted_iota(jnp.int32, (BT, T), 0)
    cols = jax.lax.broadcasted_iota(jnp.int32, (BT, T), 1)
    causal = cols <= rows
    s_ref[...] = jnp.where(causal, s_ref[...], NEG)

    for _ in range(K):
        s = s_ref[...]
        m = jnp.max(s, axis=1, keepdims=True)
        s_ref[...] = jnp.where(s == m, NEG, s)

    s_ref[...] = jnp.where(causal & (s_ref[...] == NEG), 1.0, 0.0)

    m_ref[...] = jnp.zeros((BT, D), jnp.float32)

    def agg(j, carry):
        ablk = s_ref[:, pl.ds(j * BT, BT)]
        hblk = h_ref[0, pl.ds(j * BT, BT), :]
        m_ref[...] += jax.lax.dot_general(
            ablk, hblk, (((1,), (0,)), ((), ())),
            preferred_element_type=jnp.float32)
        return carry

    jax.lax.fori_loop(0, i + 1, agg, 0)

    row1 = i * BT + jax.lax.broadcasted_iota(jnp.int32, (BT, 1), 0)
    deg = jnp.minimum(row1.astype(jnp.float32) + 1.0, float(K))
    msg = m_ref[...] * (1.0 / deg)

    blended = mix * q + (1.0 - mix) * msg
    gb = blended * gain_ref[0] + bias_ref[0]
    act = gb * 0.5 * (1.0 + jax.lax.erf(gb * (1.0 / math.sqrt(2.0))))
    hn = momentum * q + (1.0 - momentum) * act
    if is_last:
        out_ref[0] = (hn - x_ref[0, pl.ds(i * BT, BT), :]) * scale
    else:
        out_ref[0] = hn


def _round_call(r, is_last, h, x, gain_r, bias_r, params, BT=1024):
    B, T, D = h.shape
    NI = T // BT
    body = functools.partial(_round_body, r, is_last, BT, T, D, NI)
    in_specs = [
        pl.BlockSpec(memory_space=pltpu.SMEM),
        pl.BlockSpec((1, T, D), lambda b, i: (b, 0, 0)),
    ]
    args = [params, h]
    if is_last:
        in_specs.append(pl.BlockSpec((1, T, D), lambda b, i: (b, 0, 0)))
        args.append(x)
    in_specs += [
        pl.BlockSpec((1, D), lambda b, i: (0, 0)),
        pl.BlockSpec((1, D), lambda b, i: (0, 0)),
    ]
    args += [gain_r, bias_r]
    return pl.pallas_call(
        body,
        grid=(B, NI),
        in_specs=in_specs,
        out_specs=pl.BlockSpec((1, BT, D), lambda b, i: (b, i, 0)),
        out_shape=jax.ShapeDtypeStruct((B, T, D), jnp.float32),
        scratch_shapes=[
            pltpu.VMEM((BT, T), jnp.float32),
            pltpu.VMEM((BT, D), jnp.float32),
        ],
    )(*args)


def kernel(x, gain, bias, log_mix, log_momentum, log_scale):
    B, T, D = x.shape
    momentum = jax.nn.sigmoid(log_momentum)
    scale = jax.nn.softplus(log_scale) + 0.01
    mix = jax.nn.sigmoid(log_mix)
    params = jnp.concatenate(
        [mix.astype(jnp.float32),
         jnp.stack([momentum, scale]).astype(jnp.float32)])
    h = x
    for r in range(R):
        h = _round_call(r, r == R - 1, h, x,
                        gain[r].reshape(1, D), bias[r].reshape(1, D), params)
    return h


# speedup vs baseline: 1.7069x; 1.0014x over previous
import functools
import math

import jax
import jax.numpy as jnp
from jax.experimental import pallas as pl
from jax.experimental.pallas import tpu as pltpu

K = 8
R = 3
NEG = -1e38


def _round_body(r, is_last, BT, T, D, NI,
                params_ref, h_ref, *rest):
    if is_last:
        x_ref, gain_ref, bias_ref, out_ref, s_ref, m_ref = rest
    else:
        gain_ref, bias_ref, out_ref, s_ref, m_ref = rest
    i = pl.program_id(1)
    mix = params_ref[r]
    momentum = params_ref[R]
    scale = params_ref[R + 1]

    q = h_ref[0, pl.ds(i * BT, BT), :]

    def fill(j, carry):
        kblk = h_ref[0, pl.ds(j * BT, BT), :]
        s_ref[:, pl.ds(j * BT, BT)] = jax.lax.dot_general(
            q, kblk, (((1,), (1,)), ((), ())),
            preferred_element_type=jnp.float32)
        return carry

    jax.lax.fori_loop(0, i + 1, fill, 0)

    def select_at_width(W):
        def go():
            rows = i * BT + jax.lax.broadcasted_iota(jnp.int32, (BT, W), 0)
            cols = jax.lax.broadcasted_iota(jnp.int32, (BT, W), 1)
            causal = cols <= rows
            s_ref[:, :W] = jnp.where(causal, s_ref[:, :W], NEG)
            for _ in range(K):
                s = s_ref[:, :W]
                m = jnp.max(s, axis=1, keepdims=True)
                s_ref[:, :W] = jnp.where(s == m, NEG, s)
            s_ref[:, :W] = jnp.where(causal & (s_ref[:, :W] == NEG),
                                     1.0, 0.0)
        return go

    for ii in range(NI):
        pl.when(i == ii)(select_at_width((ii + 1) * BT))

    m_ref[...] = jnp.zeros((BT, D), jnp.float32)

    def agg(j, carry):
        ablk = s_ref[:, pl.ds(j * BT, BT)]
        hblk = h_ref[0, pl.ds(j * BT, BT), :]
        m_ref[...] += jax.lax.dot_general(
            ablk, hblk, (((1,), (0,)), ((), ())),
            preferred_element_type=jnp.float32)
        return carry

    jax.lax.fori_loop(0, i + 1, agg, 0)

    row1 = i * BT + jax.lax.broadcasted_iota(jnp.int32, (BT, 1), 0)
    deg = jnp.minimum(row1.astype(jnp.float32) + 1.0, float(K))
    msg = m_ref[...] * (1.0 / deg)

    blended = mix * q + (1.0 - mix) * msg
    gb = blended * gain_ref[0] + bias_ref[0]
    act = gb * 0.5 * (1.0 + jax.lax.erf(gb * (1.0 / math.sqrt(2.0))))
    hn = momentum * q + (1.0 - momentum) * act
    if is_last:
        out_ref[0] = (hn - x_ref[0, pl.ds(i * BT, BT), :]) * scale
    else:
        out_ref[0] = hn


def _round_call(r, is_last, h, x, gain_r, bias_r, params, BT=1024):
    B, T, D = h.shape
    BT = min(BT, T)
    NI = T // BT
    body = functools.partial(_round_body, r, is_last, BT, T, D, NI)
    in_specs = [
        pl.BlockSpec(memory_space=pltpu.SMEM),
        pl.BlockSpec((1, T, D), lambda b, i: (b, 0, 0)),
    ]
    args = [params, h]
    if is_last:
        in_specs.append(pl.BlockSpec((1, T, D), lambda b, i: (b, 0, 0)))
        args.append(x)
    in_specs += [
        pl.BlockSpec((1, D), lambda b, i: (0, 0)),
        pl.BlockSpec((1, D), lambda b, i: (0, 0)),
    ]
    args += [gain_r, bias_r]
    return pl.pallas_call(
        body,
        grid=(B, NI),
        in_specs=in_specs,
        out_specs=pl.BlockSpec((1, BT, D), lambda b, i: (b, i, 0)),
        out_shape=jax.ShapeDtypeStruct((B, T, D), jnp.float32),
        scratch_shapes=[
            pltpu.VMEM((BT, T), jnp.float32),
            pltpu.VMEM((BT, D), jnp.float32),
        ],
    )(*args)


def kernel(x, gain, bias, log_mix, log_momentum, log_scale):
    B, T, D = x.shape
    momentum = jax.nn.sigmoid(log_momentum)
    scale = jax.nn.softplus(log_scale) + 0.01
    mix = jax.nn.sigmoid(log_mix)
    params = jnp.concatenate(
        [mix.astype(jnp.float32),
         jnp.stack([momentum, scale]).astype(jnp.float32)])
    h = x
    for r in range(R):
        h = _round_call(r, r == R - 1, h, x,
                        gain[r].reshape(1, D), bias[r].reshape(1, D), params)
    return h
